# baseline (speedup 1.0000x reference)
import functools

import jax
import jax.numpy as jnp
from jax import lax
from jax.experimental import pallas as pl
from jax.experimental.pallas import tpu as pltpu
from jax.experimental.pallas import tpu_sc as plsc

_NC = 2
_NS = 16
_NW = _NC * _NS
_L = 16


@functools.partial(jax.jit, static_argnums=(3, 4, 5, 6))
def _gather_mul(idx, wt, words, B, F, H, C):
  R = B * F
  cpf = B // C
  nch = R // (C * _NW)
  mesh = plsc.VectorSubcoreMesh(core_axis_name="c", subcore_axis_name="s")

  @functools.partial(
      pl.kernel,
      out_type=jax.ShapeDtypeStruct((F * H * B,), jnp.float32),
      mesh=mesh,
      scratch_types=[
          [pltpu.VMEM((C,), jnp.int32)] * 2,
          [pltpu.VMEM((H * C,), jnp.float32)] * 2,
          [pltpu.VMEM((C,), jnp.int32)] * 2,
          [pltpu.VMEM((H * C,), jnp.float32)] * 2,
          [pltpu.SemaphoreType.DMA] * 2,
          [pltpu.SemaphoreType.DMA] * 2,
          [pltpu.SemaphoreType.DMA] * 2,
      ],
      compiler_params=pltpu.CompilerParams(
          needs_layout_passes=False, use_tc_tiling_on_sc=False
      ),
  )
  def gk(idx_hbm, wt_hbm, mw_hbm, out_hbm, idx_v, g_v, m_v, col_v, sem_w,
         sem_m, sem_o):
    wid = lax.axis_index("s") * _NC + lax.axis_index("c")
    lanes = lax.iota(jnp.int32, _L)
    q0 = wid * nch

    def start_gathers(c, p):
      pltpu.sync_copy(idx_hbm.at[pl.ds((q0 + c) * C, C)], idx_v[p])
      for h in range(H):
        pltpu.async_copy(
            wt_hbm.at[h].at[idx_v[p]], g_v[p].at[pl.ds(h * C, C)], sem_w[p]
        )
      pltpu.async_copy(mw_hbm.at[idx_v[p]], m_v[p], sem_m[p])

    def wait_gathers(p):
      for h in range(H):
        pltpu.make_async_copy(
            wt_hbm.at[h].at[idx_v[p]], g_v[p].at[pl.ds(h * C, C)], sem_w[p]
        ).wait()
      pltpu.make_async_copy(mw_hbm.at[idx_v[p]], m_v[p], sem_m[p]).wait()

    def drain_out(p):
      for h in range(H):
        pltpu.make_async_copy(
            col_v[p].at[pl.ds(h * C, C)],
            out_hbm.at[pl.ds(h * B, C)],
            sem_o[p],
        ).wait()

    def compute(c, p):
      @plsc.parallel_loop(0, C // _L, 1, unroll=2)
      def _grp(g):
        aux = m_v[p][pl.ds(g * _L, _L)]
        for h in range(H):
          v = g_v[p][pl.ds(h * C + g * _L, _L)]
          bits = (aux >> h) & 1
          col_v[p][pl.ds(h * C + g * _L, _L)] = v * bits.astype(jnp.float32)

      q = q0 + c
      f = q // cpf
      b0 = (q % cpf) * C
      obase = f * (H * B) + b0
      for h in range(H):
        pltpu.async_copy(
            col_v[p].at[pl.ds(h * C, C)],
            out_hbm.at[pl.ds(obase + h * B, C)],
            sem_o[p],
        )

    assert nch % 2 == 0 and nch >= 4
    start_gathers(0, 0)
    start_gathers(1, 1)
    wait_gathers(0)
    compute(0, 0)
    start_gathers(2, 0)
    wait_gathers(1)
    compute(1, 1)
    start_gathers(3, 1)

    def pair(i, carry):
      cA = 2 * i
      wait_gathers(0)
      drain_out(0)
      compute(cA, 0)
      start_gathers(cA + 2, 0)
      wait_gathers(1)
      drain_out(1)
      compute(cA + 1, 1)
      start_gathers(cA + 3, 1)
      return carry

    lax.fori_loop(1, nch // 2 - 1, pair, 0)

    wait_gathers(0)
    drain_out(0)
    compute(nch - 2, 0)
    wait_gathers(1)
    drain_out(1)
    compute(nch - 1, 1)
    drain_out(0)
    drain_out(1)

  return gk(idx, wt, words)


def kernel(x, weight, mask):
  B, F = x.shape
  V, H = weight.shape
  idx = jnp.swapaxes(x, 0, 1).reshape(B * F).astype(jnp.int32)
  pow2 = jnp.asarray([float(1 << i) for i in range(H)], dtype=jnp.float32)
  words = jnp.dot(mask.astype(jnp.float32), pow2).astype(jnp.int32)
  wt = jnp.swapaxes(weight, 0, 1)
  flat = _gather_mul(idx, wt, words, B, F, H, 512)
  return flat.reshape(F, H, B).transpose(2, 0, 1)


# speedup vs baseline: 2.6797x; 2.6797x over previous
import functools

import jax
import jax.numpy as jnp
from jax import lax
from jax.experimental import pallas as pl
from jax.experimental.pallas import tpu as pltpu
from jax.experimental.pallas import tpu_sc as plsc

_NC = 2
_NS = 16
_NW = _NC * _NS
_L = 16


@functools.partial(jax.jit, static_argnums=(3, 4, 5, 6))
def _gather_mul(idx, weight, words, B, F, H, C):
  R = B * F
  cpf = B // C
  nch = R // (C * _NW)
  mesh = plsc.VectorSubcoreMesh(core_axis_name="c", subcore_axis_name="s")

  @functools.partial(
      pl.kernel,
      out_type=jax.ShapeDtypeStruct((F * H * B,), jnp.float32),
      mesh=mesh,
      scratch_types=[
          [pltpu.VMEM((C,), jnp.int32)] * 2,
          [pltpu.VMEM((C, H), jnp.float32)] * 2,
          [pltpu.VMEM((C,), jnp.int32)] * 2,
          [pltpu.VMEM((H * C,), jnp.float32)] * 2,
          [pltpu.SemaphoreType.DMA] * 2,
          [pltpu.SemaphoreType.DMA] * 2,
          [pltpu.SemaphoreType.DMA] * 2,
      ],
      compiler_params=pltpu.CompilerParams(
          needs_layout_passes=False, use_tc_tiling_on_sc=False
      ),
  )
  def gk(idx_hbm, w_hbm, mw_hbm, out_hbm, idx_v, w_v, m_v, col_v, sem_w,
         sem_m, sem_o):
    wid = lax.axis_index("s") * _NC + lax.axis_index("c")
    lanes = lax.iota(jnp.int32, _L)
    lane_base = lanes * C
    q0 = wid * nch

    def start_gathers(c):
      p = c % 2
      pltpu.sync_copy(idx_hbm.at[pl.ds((q0 + c) * C, C)], idx_v[p])
      cp_w = pltpu.async_copy(w_hbm.at[idx_v[p]], w_v[p], sem_w[p])
      cp_m = pltpu.async_copy(mw_hbm.at[idx_v[p]], m_v[p], sem_m[p])
      return cp_w, cp_m

    pending_gather = {0: start_gathers(0)}
    pending_out = {}

    for c in range(nch):
      p = c % 2
      if c + 1 < nch:
        pending_gather[c + 1] = start_gathers(c + 1)
      cp_w, cp_m = pending_gather.pop(c)
      cp_w.wait()
      cp_m.wait()
      if c - 2 in pending_out:
        for cp in pending_out.pop(c - 2):
          cp.wait()

      @plsc.parallel_loop(0, C, 1, unroll=4)
      def _row(j):
        word = plsc.load_gather(m_v[p], [jnp.full((_L,), j, jnp.int32)])
        bits = (word >> lanes) & 1
        val = w_v[p][j] * bits.astype(jnp.float32)
        plsc.store_scatter(col_v[p], [lane_base + j], val)

      q = q0 + c
      f = q // cpf
      b0 = (q % cpf) * C
      obase = f * (H * B) + b0
      pending_out[c] = [
          pltpu.async_copy(
              col_v[p].at[pl.ds(h * C, C)],
              out_hbm.at[pl.ds(obase + h * B, C)],
              sem_o[p],
          )
          for h in range(H)
      ]
    for cps in pending_out.values():
      for cp in cps:
        cp.wait()

  return gk(idx, weight, words)


def kernel(x, weight, mask):
  B, F = x.shape
  V, H = weight.shape
  idx = jnp.swapaxes(x, 0, 1).reshape(B * F).astype(jnp.int32)
  pow2 = jnp.asarray([float(1 << i) for i in range(H)], dtype=jnp.float32)
  words = jnp.dot(mask.astype(jnp.float32), pow2).astype(jnp.int32)
  flat = _gather_mul(idx, weight, words, B, F, H, 1024)
  return flat.reshape(F, H, B).transpose(2, 0, 1)
